# baseline (speedup 1.0000x reference)
import functools

import jax
import jax.numpy as jnp
from jax import lax
from jax.experimental import pallas as pl
from jax.experimental.pallas import tpu as pltpu, tpu_sc as plsc

BATCH = 16384
DIM = 64
NC = 2
NS = 16
NW = NC * NS
BPW = BATCH // NW

_mesh = plsc.VectorSubcoreMesh(core_axis_name="c", subcore_axis_name="s")


@functools.partial(
    pl.kernel,
    mesh=_mesh,
    out_type=[
        jax.ShapeDtypeStruct((BATCH, DIM), jnp.float32),
        jax.ShapeDtypeStruct((BATCH, DIM), jnp.float32),
    ],
    scratch_types=[
        pltpu.VMEM((BPW,), jnp.int32),
        pltpu.VMEM((BPW,), jnp.int32),
        pltpu.VMEM((BPW, DIM), jnp.float32),
        pltpu.VMEM((BPW, DIM), jnp.float32),
        pltpu.SemaphoreType.DMA,
        pltpu.SemaphoreType.DMA,
    ],
    compiler_params=pltpu.CompilerParams(use_tc_tiling_on_sc=False),
)
def _embed(uids_hbm, pids_hbm, utab_hbm, ptab_hbm, uout_hbm, pout_hbm,
           uidx_v, pidx_v, urows_v, prows_v, usem, psem):
    wid = lax.axis_index("s") * NC + lax.axis_index("c")
    base = wid * BPW
    pltpu.sync_copy(uids_hbm.at[pl.ds(base, BPW)], uidx_v)
    pltpu.sync_copy(pids_hbm.at[pl.ds(base, BPW)], pidx_v)
    ucopy = pltpu.async_copy(utab_hbm.at[uidx_v], urows_v, usem)
    pcopy = pltpu.async_copy(ptab_hbm.at[pidx_v], prows_v, psem)
    ucopy.wait()
    pltpu.sync_copy(urows_v, uout_hbm.at[pl.ds(base, BPW)])
    pcopy.wait()
    pltpu.sync_copy(prows_v, pout_hbm.at[pl.ds(base, BPW)])


def kernel(user_ids, product_ids, user_table, product_table):
    u, p = _embed(user_ids, product_ids, user_table, product_table)
    return (u, p)


# speedup vs baseline: 1.6024x; 1.6024x over previous
import functools

import jax
import jax.numpy as jnp
from jax import lax
from jax.experimental import pallas as pl
from jax.experimental.pallas import tpu as pltpu, tpu_sc as plsc

BATCH = 16384
DIM = 64
NC = 2
NS = 16
NW = NC * NS
BPW = BATCH // NW
GRP = 4
NGRP = BPW // GRP
WCHUNK = 128
NWCHUNK = BPW // WCHUNK

_mesh = plsc.VectorSubcoreMesh(core_axis_name="c", subcore_axis_name="s")


@functools.partial(
    pl.kernel,
    mesh=_mesh,
    out_type=jax.ShapeDtypeStruct((BATCH, DIM), jnp.float32),
    scratch_types=[
        pltpu.SMEM((BPW,), jnp.int32),
        pltpu.VMEM((BPW,), jnp.int32),
        pltpu.VMEM((BPW, DIM), jnp.float32),
        pltpu.SemaphoreType.DMA,
        pltpu.SemaphoreType.DMA,
    ],
)
def _embed_one(ids_hbm, tab_hbm, out_hbm, idx_s, idx_v, rows_v, gsem, wsem):
    wid = lax.axis_index("s") * NC + lax.axis_index("c")
    base = wid * BPW

    pltpu.sync_copy(ids_hbm.at[pl.ds(base, BPW)], idx_v)

    @pl.loop(0, BPW // 16)
    def spill_loop(g):
        v = idx_v[pl.ds(g * 16, 16)]
        for u in range(16):
            idx_s[g * 16 + u] = v[u]

    @pl.loop(0, NGRP)
    def fire_loop(g):
        for u in range(GRP):
            i = g * GRP + u
            pltpu.async_copy(
                tab_hbm.at[pl.ds(idx_s[i], 1)],
                rows_v.at[pl.ds(i, 1)],
                gsem,
            )

    pltpu.make_async_copy(tab_hbm.at[pl.ds(0, BPW)], rows_v, gsem).wait()

    for c in range(NWCHUNK):
        pltpu.async_copy(
            rows_v.at[pl.ds(c * WCHUNK, WCHUNK)],
            out_hbm.at[pl.ds(base + c * WCHUNK, WCHUNK)],
            wsem,
        )
    pltpu.make_async_copy(rows_v, out_hbm.at[pl.ds(base, BPW)], wsem).wait()


def kernel(user_ids, product_ids, user_table, product_table):
    u = _embed_one(user_ids.astype(jnp.int32), user_table)
    p = _embed_one(product_ids.astype(jnp.int32), product_table)
    return (u, p)
